# baseline (speedup 1.0000x reference)
import functools

import jax
import jax.numpy as jnp
from jax import lax
from jax.experimental import pallas as pl
from jax.experimental.pallas import tpu as pltpu
from jax.experimental.pallas import tpu_sc as plsc

N = 10000
E = 160000
D_IN = 256
D_OUT = 512
H = D_IN // 2
EPS = 1e-5

NC = 2
NS = 16
LANES = 16

HPAD = 6144
EDGES_PER_TILE = E // NS
HIST_ITERS = EDGES_PER_TILE // LANES
RED = HPAD // NS


def _deg_body(dst_hbm, degp_hbm, dstloc, hist, buf2, accv, sem, shared):
    c = lax.axis_index("c")
    s = lax.axis_index("s")
    lo = c * (N // NC)

    pltpu.async_copy(dst_hbm.at[s], dstloc, sem).wait()

    def zero(i, _):
        hist[pl.ds(i * LANES, LANES)] = jnp.zeros((LANES,), jnp.float32)
        return _
    lax.fori_loop(0, HPAD // LANES, zero, None)

    ones = jnp.full((LANES,), 1.0, jnp.float32)

    def scat(i, _):
        idx = dstloc[pl.ds(i * LANES, LANES)]
        inb = (idx >= lo) & (idx < lo + (N // NC))
        il = jnp.minimum(jnp.maximum(idx - lo, 0), HPAD - 1)
        plsc.addupdate_scatter(hist, [il], ones, mask=inb)
        return _
    lax.fori_loop(0, HIST_ITERS, scat, None)

    pltpu.sync_copy(hist, shared.at[s])
    plsc.subcore_barrier()
    pltpu.sync_copy(shared.at[:, pl.ds(s * RED, RED)], buf2)

    def zacc(k, _):
        accv[pl.ds(k * LANES, LANES)] = jnp.zeros((LANES,), jnp.float32)
        return _
    lax.fori_loop(0, RED // LANES, zacc, None)

    def red_j(j, _):
        def red_k(k, __):
            sl = pl.ds(k * LANES, LANES)
            accv[sl] = accv[sl] + buf2[j, sl]
            return __
        lax.fori_loop(0, RED // LANES, red_k, None)
        return _
    lax.fori_loop(0, NS, red_j, None)

    pltpu.sync_copy(accv, degp_hbm.at[c, pl.ds(s * RED, RED)])


def _make_deg_kernel():
    mesh = plsc.VectorSubcoreMesh(core_axis_name="c", subcore_axis_name="s")

    return pl.kernel(
        _deg_body,
        out_type=jax.ShapeDtypeStruct((NC, HPAD), jnp.float32),
        mesh=mesh,
        scratch_types=[
            pltpu.VMEM((EDGES_PER_TILE,), jnp.int32),
            pltpu.VMEM((HPAD,), jnp.float32),
            pltpu.VMEM((NS, RED), jnp.float32),
            pltpu.VMEM((RED,), jnp.float32),
            pltpu.SemaphoreType.DMA,
            pltpu.VMEM_SHARED((NS, HPAD), jnp.float32),
        ],
        compiler_params=pltpu.CompilerParams(needs_layout_passes=False),
    )


CHUNK = 80
NCHUNK = E // (NS * CHUNK)
ROWS_PER_TILE = 624
ROWS_REM = N - NS * ROWS_PER_TILE


def _make_agg_kernel():
    mesh = plsc.VectorSubcoreMesh(core_axis_name="c", subcore_axis_name="s")

    def body(xs_hbm, src_hbm, dst_hbm, agg_hbm, srcloc, dstloc, rows0, rows1,
             sem, sem0, sem1, ssem0, ssem1, accum):
        c = lax.axis_index("c")
        s = lax.axis_index("s")

        pltpu.async_copy(src_hbm.at[s], srcloc, sem).wait()
        pltpu.async_copy(dst_hbm.at[s], dstloc, sem).wait()

        r0 = s * ROWS_PER_TILE
        pltpu.sync_copy(xs_hbm.at[c, pl.ds(r0, ROWS_PER_TILE)],
                        accum.at[pl.ds(r0, ROWS_PER_TILE)])

        @pl.when(s == NS - 1)
        def _():
            pltpu.sync_copy(xs_hbm.at[c, pl.ds(NS * ROWS_PER_TILE, ROWS_REM)],
                            accum.at[pl.ds(NS * ROWS_PER_TILE, ROWS_REM)])

        plsc.subcore_barrier()

        xs_c = xs_hbm.at[c]

        def src_ix(g):
            return srcloc.at[pl.ds(g * CHUNK, CHUNK)]

        def gather(g, buf, s_):
            pltpu.async_copy(xs_c.at[src_ix(g)], buf, s_)

        def gwait(g, buf, s_):
            pltpu.make_async_copy(xs_c.at[src_ix(g)], buf, s_).wait()

        def scat(g, buf, s_):
            pltpu.async_copy(buf, accum.at[dstloc.at[g]], s_, add=True)

        def swait(g, buf, s_):
            pltpu.make_async_copy(buf, accum.at[dstloc.at[g]], s_).wait()

        gather(0, rows0, sem0)
        gwait(0, rows0, sem0)
        gather(1, rows1, sem1)
        scat(0, rows0, ssem0)

        def step(i, _):
            g1 = 2 * i + 1
            g2 = 2 * i + 2
            g3 = jnp.minimum(2 * i + 3, NCHUNK - 1)
            gwait(g1, rows1, sem1)
            swait(g1 - 1, rows0, ssem0)
            gather(g2, rows0, sem0)
            scat(g1, rows1, ssem1)
            gwait(g2, rows0, sem0)
            swait(g1, rows1, ssem1)
            gather(g3, rows1, sem1)
            scat(g2, rows0, ssem0)
            return _
        lax.fori_loop(0, NCHUNK // 2, step, None)
        gwait(NCHUNK - 1, rows1, sem1)
        swait(NCHUNK - 1, rows0, ssem0)

        plsc.subcore_barrier()
        pltpu.sync_copy(accum.at[pl.ds(r0, ROWS_PER_TILE)],
                        agg_hbm.at[c, pl.ds(r0, ROWS_PER_TILE)])

        @pl.when(s == NS - 1)
        def _():
            pltpu.sync_copy(accum.at[pl.ds(NS * ROWS_PER_TILE, ROWS_REM)],
                            agg_hbm.at[c, pl.ds(NS * ROWS_PER_TILE, ROWS_REM)])

    return pl.kernel(
        body,
        out_type=jax.ShapeDtypeStruct((NC, N, H), jnp.float32),
        mesh=mesh,
        scratch_types=[
            pltpu.VMEM((EDGES_PER_TILE,), jnp.int32),
            pltpu.VMEM((NCHUNK, CHUNK), jnp.int32),
            pltpu.VMEM((CHUNK, H), jnp.float32),
            pltpu.VMEM((CHUNK, H), jnp.float32),
            pltpu.SemaphoreType.DMA,
            pltpu.SemaphoreType.DMA,
            pltpu.SemaphoreType.DMA,
            pltpu.SemaphoreType.DMA,
            pltpu.SemaphoreType.DMA,
            pltpu.VMEM_SHARED((N, H), jnp.float32),
        ],
    )


RB = 1000
GRID = N // RB


def _prescale_body(deg_ref, x_ref, dinv_ref, xs_ref):
    d = deg_ref[...] + 1.0
    dinv = jnp.where(d > 0.0, lax.rsqrt(d), 0.0)
    dinv_ref[...] = dinv
    xsb = x_ref[...] * dinv
    xs_ref[0] = xsb[:, :H]
    xs_ref[1] = xsb[:, H:]


def _matmul_body(agg_ref, dinv_ref, w_ref, b_ref, out_ref, stats_ref):
    aggf = jnp.concatenate([agg_ref[0], agg_ref[1]], axis=1)
    pre = aggf * dinv_ref[...]
    o = jnp.dot(pre, w_ref[...], preferred_element_type=jnp.float32)
    o = o + b_ref[...]
    out_ref[...] = o

    @pl.when(pl.program_id(0) == 0)
    def _():
        stats_ref[...] = jnp.zeros_like(stats_ref)

    stats_ref[0:1, :] += jnp.sum(o, axis=0, keepdims=True)
    stats_ref[1:2, :] += jnp.sum(o * o, axis=0, keepdims=True)


def _bn_body(out_ref, stats_ref, gamma_ref, beta_ref, y_ref):
    mu = stats_ref[0:1, :] * (1.0 / N)
    ex2 = stats_ref[1:2, :] * (1.0 / N)
    var = jnp.maximum(ex2 - mu * mu, 0.0)
    inv = lax.rsqrt(var + EPS)
    y = (out_ref[...] - mu) * (inv * gamma_ref[...]) + beta_ref[...]
    y_ref[...] = jnp.maximum(y, 0.0)


@jax.jit
def kernel(x, adj_t, W, b, gamma, beta):
    src = adj_t[0].astype(jnp.int32)
    dst = adj_t[1].astype(jnp.int32)

    degp = _make_deg_kernel()(dst.reshape(NS, EDGES_PER_TILE))
    deg2d = jnp.concatenate(
        [degp[0, : N // NC], degp[1, : N // NC]]).reshape(N, 1)

    dinv2d, xs = pl.pallas_call(
        _prescale_body,
        grid=(GRID,),
        in_specs=[
            pl.BlockSpec((RB, 1), lambda r: (r, 0)),
            pl.BlockSpec((RB, D_IN), lambda r: (r, 0)),
        ],
        out_specs=[
            pl.BlockSpec((RB, 1), lambda r: (r, 0)),
            pl.BlockSpec((NC, RB, H), lambda r: (0, r, 0)),
        ],
        out_shape=[
            jax.ShapeDtypeStruct((N, 1), jnp.float32),
            jax.ShapeDtypeStruct((NC, N, H), jnp.float32),
        ],
    )(deg2d, x)

    agg = _make_agg_kernel()(
        xs,
        src.reshape(NS, EDGES_PER_TILE),
        dst.reshape(NS, NCHUNK, CHUNK),
    )

    out_pre, stats = pl.pallas_call(
        _matmul_body,
        grid=(GRID,),
        in_specs=[
            pl.BlockSpec((NC, RB, H), lambda r: (0, r, 0)),
            pl.BlockSpec((RB, 1), lambda r: (r, 0)),
            pl.BlockSpec((D_IN, D_OUT), lambda r: (0, 0)),
            pl.BlockSpec((1, D_OUT), lambda r: (0, 0)),
        ],
        out_specs=[
            pl.BlockSpec((RB, D_OUT), lambda r: (r, 0)),
            pl.BlockSpec((2, D_OUT), lambda r: (0, 0)),
        ],
        out_shape=[
            jax.ShapeDtypeStruct((N, D_OUT), jnp.float32),
            jax.ShapeDtypeStruct((2, D_OUT), jnp.float32),
        ],
    )(agg, dinv2d, W, b.reshape(1, D_OUT))

    y = pl.pallas_call(
        _bn_body,
        grid=(GRID,),
        in_specs=[
            pl.BlockSpec((RB, D_OUT), lambda r: (r, 0)),
            pl.BlockSpec((2, D_OUT), lambda r: (0, 0)),
            pl.BlockSpec((1, D_OUT), lambda r: (0, 0)),
            pl.BlockSpec((1, D_OUT), lambda r: (0, 0)),
        ],
        out_specs=pl.BlockSpec((RB, D_OUT), lambda r: (r, 0)),
        out_shape=jax.ShapeDtypeStruct((N, D_OUT), jnp.float32),
    )(out_pre, stats, gamma.reshape(1, D_OUT), beta.reshape(1, D_OUT))

    return y


# speedup vs baseline: 1.0507x; 1.0507x over previous
import functools

import jax
import jax.numpy as jnp
from jax import lax
from jax.experimental import pallas as pl
from jax.experimental.pallas import tpu as pltpu
from jax.experimental.pallas import tpu_sc as plsc

N = 10000
E = 160000
D_IN = 256
D_OUT = 512
H = D_IN // 2
EPS = 1e-5

NC = 2
NS = 16
LANES = 16

HPAD = 6144
EDGES_PER_TILE = E // NS
HIST_ITERS = EDGES_PER_TILE // LANES
RED = HPAD // NS


def _deg_body(dst_hbm, degp_hbm, dstloc, hist, buf2, accv, sem, shared):
    c = lax.axis_index("c")
    s = lax.axis_index("s")
    lo = c * (N // NC)

    pltpu.async_copy(dst_hbm.at[s], dstloc, sem).wait()

    def zero(i, _):
        hist[pl.ds(i * LANES, LANES)] = jnp.zeros((LANES,), jnp.float32)
        return _
    lax.fori_loop(0, HPAD // LANES, zero, None)

    ones = jnp.full((LANES,), 1.0, jnp.float32)

    def scat(i, _):
        idx = dstloc[pl.ds(i * LANES, LANES)]
        inb = (idx >= lo) & (idx < lo + (N // NC))
        il = jnp.minimum(jnp.maximum(idx - lo, 0), HPAD - 1)
        plsc.addupdate_scatter(hist, [il], ones, mask=inb)
        return _
    lax.fori_loop(0, HIST_ITERS, scat, None)

    pltpu.sync_copy(hist, shared.at[s])
    plsc.subcore_barrier()
    pltpu.sync_copy(shared.at[:, pl.ds(s * RED, RED)], buf2)

    def zacc(k, _):
        accv[pl.ds(k * LANES, LANES)] = jnp.zeros((LANES,), jnp.float32)
        return _
    lax.fori_loop(0, RED // LANES, zacc, None)

    def red_j(j, _):
        def red_k(k, __):
            sl = pl.ds(k * LANES, LANES)
            accv[sl] = accv[sl] + buf2[j, sl]
            return __
        lax.fori_loop(0, RED // LANES, red_k, None)
        return _
    lax.fori_loop(0, NS, red_j, None)

    pltpu.sync_copy(accv, degp_hbm.at[c, pl.ds(s * RED, RED)])


def _make_deg_kernel():
    mesh = plsc.VectorSubcoreMesh(core_axis_name="c", subcore_axis_name="s")

    return pl.kernel(
        _deg_body,
        out_type=jax.ShapeDtypeStruct((NC, HPAD), jnp.float32),
        mesh=mesh,
        scratch_types=[
            pltpu.VMEM((EDGES_PER_TILE,), jnp.int32),
            pltpu.VMEM((HPAD,), jnp.float32),
            pltpu.VMEM((NS, RED), jnp.float32),
            pltpu.VMEM((RED,), jnp.float32),
            pltpu.SemaphoreType.DMA,
            pltpu.VMEM_SHARED((NS, HPAD), jnp.float32),
        ],
        compiler_params=pltpu.CompilerParams(needs_layout_passes=False),
    )


CHUNK = 80
NCHUNK = E // (NS * CHUNK)
ROWS_PER_TILE = 624
ROWS_REM = N - NS * ROWS_PER_TILE


def _make_agg_kernel():
    mesh = plsc.VectorSubcoreMesh(core_axis_name="c", subcore_axis_name="s")

    def body(xs_hbm, src_hbm, dst_hbm, agg_hbm, srcloc, dstloc, rows0, rows1,
             sem, sem0, sem1, ssem0, ssem1, accum):
        c = lax.axis_index("c")
        s = lax.axis_index("s")

        pltpu.async_copy(src_hbm.at[s], srcloc, sem).wait()
        pltpu.async_copy(dst_hbm.at[s], dstloc, sem).wait()

        r0 = s * ROWS_PER_TILE
        pltpu.sync_copy(xs_hbm.at[c, pl.ds(r0, ROWS_PER_TILE)],
                        accum.at[pl.ds(r0, ROWS_PER_TILE)])

        @pl.when(s == NS - 1)
        def _():
            pltpu.sync_copy(xs_hbm.at[c, pl.ds(NS * ROWS_PER_TILE, ROWS_REM)],
                            accum.at[pl.ds(NS * ROWS_PER_TILE, ROWS_REM)])

        plsc.subcore_barrier()

        xs_c = xs_hbm.at[c]

        def src_ix(g):
            return srcloc.at[pl.ds(g * CHUNK, CHUNK)]

        def gather(g, buf, s_):
            pltpu.async_copy(xs_c.at[src_ix(g)], buf, s_)

        def gwait(g, buf, s_):
            pltpu.make_async_copy(xs_c.at[src_ix(g)], buf, s_).wait()

        def scat(g, buf, s_):
            pltpu.async_copy(buf, accum.at[dstloc.at[g]], s_, add=True)

        def swait(g, buf, s_):
            pltpu.make_async_copy(buf, accum.at[dstloc.at[g]], s_).wait()

        gather(0, rows0, sem0)
        gwait(0, rows0, sem0)
        gather(1, rows1, sem1)
        scat(0, rows0, ssem0)

        def step(i, _):
            g1 = 2 * i + 1
            g2 = 2 * i + 2
            g3 = jnp.minimum(2 * i + 3, NCHUNK - 1)
            gwait(g1, rows1, sem1)
            swait(g1 - 1, rows0, ssem0)
            gather(g2, rows0, sem0)
            scat(g1, rows1, ssem1)
            gwait(g2, rows0, sem0)
            swait(g1, rows1, ssem1)
            gather(g3, rows1, sem1)
            scat(g2, rows0, ssem0)
            return _
        lax.fori_loop(0, NCHUNK // 2, step, None)
        gwait(NCHUNK - 1, rows1, sem1)
        swait(NCHUNK - 1, rows0, ssem0)

        plsc.subcore_barrier()
        pltpu.sync_copy(accum.at[pl.ds(r0, ROWS_PER_TILE)],
                        agg_hbm.at[c, pl.ds(r0, ROWS_PER_TILE)])

        @pl.when(s == NS - 1)
        def _():
            pltpu.sync_copy(accum.at[pl.ds(NS * ROWS_PER_TILE, ROWS_REM)],
                            agg_hbm.at[c, pl.ds(NS * ROWS_PER_TILE, ROWS_REM)])

    return pl.kernel(
        body,
        out_type=jax.ShapeDtypeStruct((NC, N, H), jnp.float32),
        mesh=mesh,
        scratch_types=[
            pltpu.VMEM((EDGES_PER_TILE,), jnp.int32),
            pltpu.VMEM((NCHUNK, CHUNK), jnp.int32),
            pltpu.VMEM((CHUNK, H), jnp.float32),
            pltpu.VMEM((CHUNK, H), jnp.float32),
            pltpu.SemaphoreType.DMA,
            pltpu.SemaphoreType.DMA,
            pltpu.SemaphoreType.DMA,
            pltpu.SemaphoreType.DMA,
            pltpu.SemaphoreType.DMA,
            pltpu.VMEM_SHARED((N, H), jnp.float32),
        ],
    )


RB = 1000
GRID = N // RB


def _prescale_body(deg_ref, x_ref, dinv_ref, xs_ref):
    d = deg_ref[...] + 1.0
    dinv = jnp.where(d > 0.0, lax.rsqrt(d), 0.0)
    dinv_ref[...] = dinv
    xsb = x_ref[...] * dinv
    xs_ref[0] = xsb[:, :H]
    xs_ref[1] = xsb[:, H:]


def _mm_bn_body(agg_ref, dinv_ref, w_ref, b_ref, gamma_ref, beta_ref, y_ref,
                out_buf, stats_buf):
    p = pl.program_id(0)
    r = pl.program_id(1)

    @pl.when(p == 0)
    def _():
        aggf = jnp.concatenate([agg_ref[0], agg_ref[1]], axis=1)
        pre = aggf * dinv_ref[...]
        o = jnp.dot(pre, w_ref[...], preferred_element_type=jnp.float32)
        o = o + b_ref[...]
        out_buf[pl.ds(r * RB, RB), :] = o

        @pl.when(r == 0)
        def _():
            stats_buf[...] = jnp.zeros_like(stats_buf)

        stats_buf[0:1, :] += jnp.sum(o, axis=0, keepdims=True)
        stats_buf[1:2, :] += jnp.sum(o * o, axis=0, keepdims=True)

    @pl.when(p == 1)
    def _():
        mu = stats_buf[0:1, :] * (1.0 / N)
        ex2 = stats_buf[1:2, :] * (1.0 / N)
        var = jnp.maximum(ex2 - mu * mu, 0.0)
        inv = lax.rsqrt(var + EPS)
        o = out_buf[pl.ds(r * RB, RB), :]
        y = (o - mu) * (inv * gamma_ref[...]) + beta_ref[...]
        y_ref[...] = jnp.maximum(y, 0.0)


@jax.jit
def kernel(x, adj_t, W, b, gamma, beta):
    src = adj_t[0].astype(jnp.int32)
    dst = adj_t[1].astype(jnp.int32)

    degp = _make_deg_kernel()(dst.reshape(NS, EDGES_PER_TILE))
    deg2d = jnp.concatenate(
        [degp[0, : N // NC], degp[1, : N // NC]]).reshape(N, 1)

    dinv2d, xs = pl.pallas_call(
        _prescale_body,
        grid=(GRID,),
        in_specs=[
            pl.BlockSpec((RB, 1), lambda r: (r, 0)),
            pl.BlockSpec((RB, D_IN), lambda r: (r, 0)),
        ],
        out_specs=[
            pl.BlockSpec((RB, 1), lambda r: (r, 0)),
            pl.BlockSpec((NC, RB, H), lambda r: (0, r, 0)),
        ],
        out_shape=[
            jax.ShapeDtypeStruct((N, 1), jnp.float32),
            jax.ShapeDtypeStruct((NC, N, H), jnp.float32),
        ],
    )(deg2d, x)

    agg = _make_agg_kernel()(
        xs,
        src.reshape(NS, EDGES_PER_TILE),
        dst.reshape(NS, NCHUNK, CHUNK),
    )

    y = pl.pallas_call(
        _mm_bn_body,
        grid=(2, GRID),
        in_specs=[
            pl.BlockSpec((NC, RB, H), lambda p, r: (0, jnp.where(p == 0, r, 0), 0)),
            pl.BlockSpec((RB, 1), lambda p, r: (jnp.where(p == 0, r, 0), 0)),
            pl.BlockSpec((D_IN, D_OUT), lambda p, r: (0, 0)),
            pl.BlockSpec((1, D_OUT), lambda p, r: (0, 0)),
            pl.BlockSpec((1, D_OUT), lambda p, r: (0, 0)),
            pl.BlockSpec((1, D_OUT), lambda p, r: (0, 0)),
        ],
        out_specs=pl.BlockSpec((RB, D_OUT),
                               lambda p, r: (jnp.where(p == 0, 0, r), 0)),
        out_shape=jax.ShapeDtypeStruct((N, D_OUT), jnp.float32),
        scratch_shapes=[
            pltpu.VMEM((N, D_OUT), jnp.float32),
            pltpu.VMEM((2, D_OUT), jnp.float32),
        ],
    )(agg, dinv2d, W, b.reshape(1, D_OUT), gamma.reshape(1, D_OUT),
      beta.reshape(1, D_OUT))

    return y


# speedup vs baseline: 1.0769x; 1.0249x over previous
import functools

import jax
import jax.numpy as jnp
from jax import lax
from jax.experimental import pallas as pl
from jax.experimental.pallas import tpu as pltpu
from jax.experimental.pallas import tpu_sc as plsc

N = 10000
E = 160000
D_IN = 256
D_OUT = 512
H = D_IN // 2
EPS = 1e-5

NC = 2
NS = 16
LANES = 16

HPAD = 6144
EDGES_PER_TILE = E // NS
HIST_ITERS = EDGES_PER_TILE // LANES
RED = HPAD // NS


def _deg_body(dst_hbm, degp_hbm, dstloc, hist, buf2, accv, sem, shared):
    c = lax.axis_index("c")
    s = lax.axis_index("s")
    lo = c * (N // NC)

    pltpu.async_copy(dst_hbm.at[s], dstloc, sem).wait()

    def zero(i, _):
        hist[pl.ds(i * LANES, LANES)] = jnp.zeros((LANES,), jnp.float32)
        return _
    lax.fori_loop(0, HPAD // LANES, zero, None)

    ones = jnp.full((LANES,), 1.0, jnp.float32)

    def scat(i, _):
        idx = dstloc[pl.ds(i * LANES, LANES)]
        inb = (idx >= lo) & (idx < lo + (N // NC))
        il = jnp.minimum(jnp.maximum(idx - lo, 0), HPAD - 1)
        plsc.addupdate_scatter(hist, [il], ones, mask=inb)
        return _
    lax.fori_loop(0, HIST_ITERS, scat, None)

    pltpu.sync_copy(hist, shared.at[s])
    plsc.subcore_barrier()
    pltpu.sync_copy(shared.at[:, pl.ds(s * RED, RED)], buf2)

    def zacc(k, _):
        accv[pl.ds(k * LANES, LANES)] = jnp.zeros((LANES,), jnp.float32)
        return _
    lax.fori_loop(0, RED // LANES, zacc, None)

    def red_j(j, _):
        def red_k(k, __):
            sl = pl.ds(k * LANES, LANES)
            accv[sl] = accv[sl] + buf2[j, sl]
            return __
        lax.fori_loop(0, RED // LANES, red_k, None)
        return _
    lax.fori_loop(0, NS, red_j, None)

    pltpu.sync_copy(accv, degp_hbm.at[c, pl.ds(s * RED, RED)])


def _make_deg_kernel():
    mesh = plsc.VectorSubcoreMesh(core_axis_name="c", subcore_axis_name="s")

    return pl.kernel(
        _deg_body,
        out_type=jax.ShapeDtypeStruct((NC, HPAD), jnp.float32),
        mesh=mesh,
        scratch_types=[
            pltpu.VMEM((EDGES_PER_TILE,), jnp.int32),
            pltpu.VMEM((HPAD,), jnp.float32),
            pltpu.VMEM((NS, RED), jnp.float32),
            pltpu.VMEM((RED,), jnp.float32),
            pltpu.SemaphoreType.DMA,
            pltpu.VMEM_SHARED((NS, HPAD), jnp.float32),
        ],
        compiler_params=pltpu.CompilerParams(needs_layout_passes=False),
    )


CHUNK = 80
NCHUNK = E // (NS * CHUNK)
ROWS_PER_TILE = 624
ROWS_REM = N - NS * ROWS_PER_TILE


def _make_agg_kernel():
    mesh = plsc.VectorSubcoreMesh(core_axis_name="c", subcore_axis_name="s")

    def body(xs_hbm, src_hbm, dst_hbm, agg_hbm, srcloc, dstloc, rows0, rows1,
             sem, sem0, sem1, ssem0, ssem1, accum):
        c = lax.axis_index("c")
        s = lax.axis_index("s")

        pltpu.async_copy(src_hbm.at[s], srcloc, sem).wait()
        pltpu.async_copy(dst_hbm.at[s], dstloc, sem).wait()

        r0 = s * ROWS_PER_TILE
        pltpu.sync_copy(xs_hbm.at[c, pl.ds(r0, ROWS_PER_TILE)],
                        accum.at[pl.ds(r0, ROWS_PER_TILE)])

        @pl.when(s == NS - 1)
        def _():
            pltpu.sync_copy(xs_hbm.at[c, pl.ds(NS * ROWS_PER_TILE, ROWS_REM)],
                            accum.at[pl.ds(NS * ROWS_PER_TILE, ROWS_REM)])

        plsc.subcore_barrier()

        xs_c = xs_hbm.at[c]

        def src_ix(g):
            return srcloc.at[pl.ds(g * CHUNK, CHUNK)]

        def gather(g, buf, s_):
            pltpu.async_copy(xs_c.at[src_ix(g)], buf, s_)

        def gwait(g, buf, s_):
            pltpu.make_async_copy(xs_c.at[src_ix(g)], buf, s_).wait()

        def scat(g, buf, s_):
            pltpu.async_copy(buf, accum.at[dstloc.at[g]], s_, add=True)

        def swait(g, buf, s_):
            pltpu.make_async_copy(buf, accum.at[dstloc.at[g]], s_).wait()

        gather(0, rows0, sem0)
        gwait(0, rows0, sem0)
        gather(1, rows1, sem1)
        scat(0, rows0, ssem0)

        def step(i, _):
            g1 = 2 * i + 1
            g2 = 2 * i + 2
            g3 = jnp.minimum(2 * i + 3, NCHUNK - 1)
            gwait(g1, rows1, sem1)
            swait(g1 - 1, rows0, ssem0)
            gather(g2, rows0, sem0)
            scat(g1, rows1, ssem1)
            gwait(g2, rows0, sem0)
            swait(g1, rows1, ssem1)
            gather(g3, rows1, sem1)
            scat(g2, rows0, ssem0)
            return _
        lax.fori_loop(0, NCHUNK // 2, step, None)
        gwait(NCHUNK - 1, rows1, sem1)
        swait(NCHUNK - 1, rows0, ssem0)

        plsc.subcore_barrier()
        pltpu.sync_copy(accum.at[pl.ds(r0, ROWS_PER_TILE)],
                        agg_hbm.at[c, pl.ds(r0, ROWS_PER_TILE)])

        @pl.when(s == NS - 1)
        def _():
            pltpu.sync_copy(accum.at[pl.ds(NS * ROWS_PER_TILE, ROWS_REM)],
                            agg_hbm.at[c, pl.ds(NS * ROWS_PER_TILE, ROWS_REM)])

    return pl.kernel(
        body,
        out_type=jax.ShapeDtypeStruct((NC, N, H), jnp.float32),
        mesh=mesh,
        scratch_types=[
            pltpu.VMEM((EDGES_PER_TILE,), jnp.int32),
            pltpu.VMEM((NCHUNK, CHUNK), jnp.int32),
            pltpu.VMEM((CHUNK, H), jnp.float32),
            pltpu.VMEM((CHUNK, H), jnp.float32),
            pltpu.SemaphoreType.DMA,
            pltpu.SemaphoreType.DMA,
            pltpu.SemaphoreType.DMA,
            pltpu.SemaphoreType.DMA,
            pltpu.SemaphoreType.DMA,
            pltpu.VMEM_SHARED((N, H), jnp.float32),
        ],
    )


RB = 2000
GRID = N // RB


def _prescale_body(deg_ref, x_ref, dinv_ref, xs_ref):
    d = deg_ref[...] + 1.0
    dinv = jnp.where(d > 0.0, lax.rsqrt(d), 0.0)
    dinv_ref[...] = dinv
    xsb = x_ref[...] * dinv
    xs_ref[0] = xsb[:, :H]
    xs_ref[1] = xsb[:, H:]


def _mm_bn_body(agg_ref, dinv_ref, w_ref, b_ref, gamma_ref, beta_ref, y_ref,
                out_buf, stats_buf):
    p = pl.program_id(0)
    r = pl.program_id(1)

    @pl.when(p == 0)
    def _():
        aggf = jnp.concatenate([agg_ref[0], agg_ref[1]], axis=1)
        pre = aggf * dinv_ref[...]
        o = jnp.dot(pre, w_ref[...], preferred_element_type=jnp.float32)
        o = o + b_ref[...]
        out_buf[pl.ds(r * RB, RB), :] = o

        @pl.when(r == 0)
        def _():
            stats_buf[...] = jnp.zeros_like(stats_buf)

        stats_buf[0:1, :] += jnp.sum(o, axis=0, keepdims=True)
        stats_buf[1:2, :] += jnp.sum(o * o, axis=0, keepdims=True)

    @pl.when(p == 1)
    def _():
        mu = stats_buf[0:1, :] * (1.0 / N)
        ex2 = stats_buf[1:2, :] * (1.0 / N)
        var = jnp.maximum(ex2 - mu * mu, 0.0)
        inv = lax.rsqrt(var + EPS)
        o = out_buf[pl.ds(r * RB, RB), :]
        y = (o - mu) * (inv * gamma_ref[...]) + beta_ref[...]
        y_ref[...] = jnp.maximum(y, 0.0)


@jax.jit
def kernel(x, adj_t, W, b, gamma, beta):
    src = adj_t[0].astype(jnp.int32)
    dst = adj_t[1].astype(jnp.int32)

    degp = _make_deg_kernel()(dst.reshape(NS, EDGES_PER_TILE))
    deg2d = jnp.concatenate(
        [degp[0, : N // NC], degp[1, : N // NC]]).reshape(N, 1)

    dinv2d, xs = pl.pallas_call(
        _prescale_body,
        grid=(GRID,),
        in_specs=[
            pl.BlockSpec((RB, 1), lambda r: (r, 0)),
            pl.BlockSpec((RB, D_IN), lambda r: (r, 0)),
        ],
        out_specs=[
            pl.BlockSpec((RB, 1), lambda r: (r, 0)),
            pl.BlockSpec((NC, RB, H), lambda r: (0, r, 0)),
        ],
        out_shape=[
            jax.ShapeDtypeStruct((N, 1), jnp.float32),
            jax.ShapeDtypeStruct((NC, N, H), jnp.float32),
        ],
    )(deg2d, x)

    agg = _make_agg_kernel()(
        xs,
        src.reshape(NS, EDGES_PER_TILE),
        dst.reshape(NS, NCHUNK, CHUNK),
    )

    y = pl.pallas_call(
        _mm_bn_body,
        grid=(2, GRID),
        in_specs=[
            pl.BlockSpec((NC, RB, H), lambda p, r: (0, jnp.where(p == 0, r, 0), 0)),
            pl.BlockSpec((RB, 1), lambda p, r: (jnp.where(p == 0, r, 0), 0)),
            pl.BlockSpec((D_IN, D_OUT), lambda p, r: (0, 0)),
            pl.BlockSpec((1, D_OUT), lambda p, r: (0, 0)),
            pl.BlockSpec((1, D_OUT), lambda p, r: (0, 0)),
            pl.BlockSpec((1, D_OUT), lambda p, r: (0, 0)),
        ],
        out_specs=pl.BlockSpec((RB, D_OUT),
                               lambda p, r: (jnp.where(p == 0, 0, r), 0)),
        out_shape=jax.ShapeDtypeStruct((N, D_OUT), jnp.float32),
        scratch_shapes=[
            pltpu.VMEM((N, D_OUT), jnp.float32),
            pltpu.VMEM((2, D_OUT), jnp.float32),
        ],
    )(agg, dinv2d, W, b.reshape(1, D_OUT), gamma.reshape(1, D_OUT),
      beta.reshape(1, D_OUT))

    return y
